# baseline (speedup 1.0000x reference)
import jax
import jax.numpy as jnp
from jax.experimental import pallas as pl
from jax.experimental.pallas import tpu as pltpu

NUM_LABELS = 1000
LATENT_DIM = 512
TOPK = 10
BLOCK_B = 512

_NEG_INF = float("-inf")


def _lwal_block(z_ref, e_ref, esq_ref, logits_ref, vals_ref, idx_ref,
                sa_ref, sb_ref):
    i = pl.program_id(0)

    def _matmul(scr_ref):
        z = z_ref[...]
        e = e_ref[...]
        z_sq = jnp.sum(z * z, axis=1, keepdims=True)
        e_sq = esq_ref[...]
        dots = jax.lax.dot_general(
            z, e, (((1,), (1,)), ((), ())),
            preferred_element_type=jnp.float32)
        logits = 2.0 * dots - z_sq - e_sq
        logits_ref[...] = logits
        scr_ref[...] = logits

    def _topk(scr_ref):
        acc = scr_ref[...]
        bb = acc.shape[0]
        fiota = jax.lax.broadcasted_iota(
            jnp.int32, (bb, NUM_LABELS), 1).astype(jnp.float32)
        for k in range(TOPK):
            m = jnp.max(acc, axis=1, keepdims=True)
            is_max = acc == m
            arg = jnp.min(jnp.where(is_max, fiota, 1024.0), axis=1,
                          keepdims=True)
            vals_ref[:, k] = m[:, 0]
            idx_ref[:, k] = arg[:, 0].astype(jnp.int32)
            acc = jnp.where(fiota == arg, _NEG_INF, acc)

    even = jax.lax.rem(i, 2) == 0

    @pl.when(even)
    def _even_step():
        _matmul(sa_ref)
        _topk(sb_ref)

    @pl.when(jnp.logical_not(even))
    def _odd_step():
        _matmul(sb_ref)
        _topk(sa_ref)


@jax.jit
def kernel(z, label_emb):
    batch = z.shape[0]
    n_blocks = batch // BLOCK_B
    e_sq = jnp.sum(label_emb * label_emb, axis=1)[None, :]

    grid = (n_blocks + 1,)
    out_shapes = (
        jax.ShapeDtypeStruct((batch, NUM_LABELS), jnp.float32),
        jax.ShapeDtypeStruct((batch, TOPK), jnp.float32),
        jax.ShapeDtypeStruct((batch, TOPK), jnp.int32),
    )
    nb = n_blocks
    logits, vals, idx = pl.pallas_call(
        _lwal_block,
        grid=grid,
        in_specs=[
            pl.BlockSpec((BLOCK_B, LATENT_DIM),
                         lambda i: (jnp.minimum(i, nb - 1), 0)),
            pl.BlockSpec((NUM_LABELS, LATENT_DIM), lambda i: (0, 0)),
            pl.BlockSpec((1, NUM_LABELS), lambda i: (0, 0)),
        ],
        out_specs=(
            pl.BlockSpec((BLOCK_B, NUM_LABELS),
                         lambda i: (jnp.minimum(i, nb - 1), 0)),
            pl.BlockSpec((BLOCK_B, TOPK),
                         lambda i: (jnp.maximum(i - 1, 0), 0)),
            pl.BlockSpec((BLOCK_B, TOPK),
                         lambda i: (jnp.maximum(i - 1, 0), 0)),
        ),
        out_shape=out_shapes,
        scratch_shapes=[pltpu.VMEM((BLOCK_B, NUM_LABELS), jnp.float32),
                        pltpu.VMEM((BLOCK_B, NUM_LABELS), jnp.float32)],
        compiler_params=pltpu.CompilerParams(
            dimension_semantics=("arbitrary",)),
    )(z, label_emb, e_sq)
    return logits, vals, idx


# speedup vs baseline: 1.0308x; 1.0308x over previous
import jax
import jax.numpy as jnp
from jax.experimental import pallas as pl
from jax.experimental.pallas import tpu as pltpu

NUM_LABELS = 1000
LATENT_DIM = 512
TOPK = 10
BLOCK_B = 1024

_NEG_INF = float("-inf")


def _lwal_block(z_ref, e_ref, esq_ref, logits_ref, vals_ref, idx_ref):
    z = z_ref[...]
    e = e_ref[...]
    z_sq = jnp.sum(z * z, axis=1, keepdims=True)
    e_sq = esq_ref[...]
    dots = jax.lax.dot_general(
        z, e, (((1,), (1,)), ((), ())), preferred_element_type=jnp.float32
    )
    logits = 2.0 * dots - z_sq - e_sq
    logits_ref[...] = logits

    bb = logits.shape[0]
    fiota = jax.lax.broadcasted_iota(
        jnp.int32, (bb, NUM_LABELS), 1).astype(jnp.float32)
    acc = logits
    for k in range(TOPK):
        m = jnp.max(acc, axis=1, keepdims=True)
        is_max = acc == m
        arg = jnp.min(jnp.where(is_max, fiota, 1024.0), axis=1,
                      keepdims=True)
        vals_ref[:, k] = m[:, 0]
        idx_ref[:, k] = arg[:, 0].astype(jnp.int32)
        acc = jnp.where(fiota == arg, _NEG_INF, acc)


@jax.jit
def kernel(z, label_emb):
    batch = z.shape[0]
    n_blocks = batch // BLOCK_B
    e_sq = jnp.sum(label_emb * label_emb, axis=1)[None, :]

    grid = (n_blocks,)
    out_shapes = (
        jax.ShapeDtypeStruct((batch, NUM_LABELS), jnp.float32),
        jax.ShapeDtypeStruct((batch, TOPK), jnp.float32),
        jax.ShapeDtypeStruct((batch, TOPK), jnp.int32),
    )
    logits, vals, idx = pl.pallas_call(
        _lwal_block,
        grid=grid,
        in_specs=[
            pl.BlockSpec((BLOCK_B, LATENT_DIM), lambda i: (i, 0)),
            pl.BlockSpec((NUM_LABELS, LATENT_DIM), lambda i: (0, 0)),
            pl.BlockSpec((1, NUM_LABELS), lambda i: (0, 0)),
        ],
        out_specs=(
            pl.BlockSpec((BLOCK_B, NUM_LABELS), lambda i: (i, 0)),
            pl.BlockSpec((BLOCK_B, TOPK), lambda i: (i, 0)),
            pl.BlockSpec((BLOCK_B, TOPK), lambda i: (i, 0)),
        ),
        out_shape=out_shapes,
    )(z, label_emb, e_sq)
    return logits, vals, idx
